# baseline (speedup 1.0000x reference)
import functools

import jax
import jax.numpy as jnp
from jax import lax
from jax.experimental import pallas as pl
from jax.experimental.pallas import tpu as pltpu
from jax.experimental.pallas import tpu_sc as plsc

_B = 4
_ROWS = 8192
_COLS = 2048
_RT = 7680
_TC_BLK = 512

_NW = 32
_WORKERS_PER_SAMPLE = _NW // _B
_SC_ROWS = _ROWS - _RT
_W_ROWS = _SC_ROWS // _WORKERS_PER_SAMPLE
_CH_ROWS = 8
_NCHUNK = _W_ROWS // _CH_ROWS
_STEP = 128


def _tc_body(var_ref, ab_ref, out_ref):
    j = pl.program_id(1)

    @pl.when(j == 0)
    def _init():
        out_ref[...] = jnp.zeros_like(out_ref)

    v = var_ref[0]
    a = ab_ref[0]
    d = jnp.where(a != 0, v - a, 0.0)
    dd = d * d
    p = jnp.sum(dd, axis=0).reshape(16, 128).sum(axis=0)
    out_ref[0, 0, :] += p


def _tc_loss(var, ab):
    partial = pl.pallas_call(
        _tc_body,
        grid=(_B, _RT // _TC_BLK),
        in_specs=[
            pl.BlockSpec((1, _TC_BLK, _COLS), lambda b, j: (b, j, 0)),
            pl.BlockSpec((1, _TC_BLK, _COLS), lambda b, j: (b, j, 0)),
        ],
        out_specs=pl.BlockSpec((1, 1, 128), lambda b, j: (b, 0, 0)),
        out_shape=jax.ShapeDtypeStruct((_B, 1, 128), jnp.float32),
    )(var, ab)
    return jnp.sum(partial, axis=(1, 2))


def _chunk_sum(buf_v, buf_a, acc):
    def body(j, acc):
        col = j * _STEP
        for r in range(_CH_ROWS):
            for k in range(_STEP // 16):
                v = buf_v[r, pl.ds(col + k * 16, 16)]
                a = buf_a[r, pl.ds(col + k * 16, 16)]
                d = jnp.where(a == 0.0, a, v) - a
                acc = acc + d * d
        return acc

    return lax.fori_loop(0, _COLS // _STEP, body, acc)


def _sc_loss_body(var_hbm, ab_hbm, out_hbm, vbuf, abuf, obuf, sv0, sv1, sa0, sa1):
    wid = lax.axis_index("s") * 2 + lax.axis_index("c")
    b = wid // _WORKERS_PER_SAMPLE
    row0 = _RT + (wid % _WORKERS_PER_SAMPLE) * _W_ROWS

    sems = (sv0, sv1, sa0, sa1)

    def start(chunk, slot):
        r = row0 + chunk * _CH_ROWS
        pltpu.make_async_copy(var_hbm.at[b, pl.ds(r, _CH_ROWS)], vbuf.at[slot],
                              sems[slot]).start()
        pltpu.make_async_copy(ab_hbm.at[b, pl.ds(r, _CH_ROWS)], abuf.at[slot],
                              sems[2 + slot]).start()

    def wait(chunk, slot):
        r = row0 + chunk * _CH_ROWS
        pltpu.make_async_copy(var_hbm.at[b, pl.ds(r, _CH_ROWS)], vbuf.at[slot],
                              sems[slot]).wait()
        pltpu.make_async_copy(ab_hbm.at[b, pl.ds(r, _CH_ROWS)], abuf.at[slot],
                              sems[2 + slot]).wait()

    start(0, 0)

    def outer(t, acc):
        g0 = 2 * t
        start(g0 + 1, 1)
        wait(g0, 0)
        acc = _chunk_sum(vbuf.at[0], abuf.at[0], acc)

        @pl.when(t + 1 < _NCHUNK // 2)
        def _():
            start(g0 + 2, 0)

        wait(g0 + 1, 1)
        acc = _chunk_sum(vbuf.at[1], abuf.at[1], acc)
        return acc

    acc = lax.fori_loop(0, _NCHUNK // 2, outer, jnp.zeros((16,), jnp.float32))

    zero = jnp.zeros((16,), jnp.float32)
    obuf[pl.ds(0, 16)] = acc
    for k in range(1, 8):
        obuf[pl.ds(k * 16, 16)] = zero
    pltpu.make_async_copy(obuf, out_hbm.at[wid], sv0).start()
    pltpu.make_async_copy(obuf, out_hbm.at[wid], sv0).wait()


_sc_loss = functools.partial(
    pl.kernel,
    mesh=plsc.VectorSubcoreMesh(core_axis_name="c", subcore_axis_name="s"),
    out_type=jax.ShapeDtypeStruct((_NW, 128), jnp.float32),
    scratch_types=[
        pltpu.VMEM((2, _CH_ROWS, _COLS), jnp.float32),
        pltpu.VMEM((2, _CH_ROWS, _COLS), jnp.float32),
        pltpu.VMEM((128,), jnp.float32),
        pltpu.SemaphoreType.DMA,
        pltpu.SemaphoreType.DMA,
        pltpu.SemaphoreType.DMA,
        pltpu.SemaphoreType.DMA,
    ],
    compiler_params=pltpu.CompilerParams(use_tc_tiling_on_sc=True),
)(_sc_loss_body)


def kernel(var, ab):
    sc_partial = _sc_loss(var, ab)
    tc_partial = _tc_loss(var, ab)
    sc = jnp.sum(sc_partial.reshape(_B, _WORKERS_PER_SAMPLE, 128), axis=(1, 2))
    return tc_partial + sc


# speedup vs baseline: 1.0294x; 1.0294x over previous
import functools

import jax
import jax.numpy as jnp
from jax import lax
from jax.experimental import pallas as pl
from jax.experimental.pallas import tpu as pltpu
from jax.experimental.pallas import tpu_sc as plsc

_B = 4
_ROWS = 8192
_COLS = 2048
_RT = 4608
_TC_BLK = 512

_NW = 32
_WORKERS_PER_SAMPLE = _NW // _B
_SC_ROWS = _ROWS - _RT
_W_ROWS = _SC_ROWS // _WORKERS_PER_SAMPLE
_CH_ROWS = 8
_NCHUNK = _W_ROWS // _CH_ROWS
_STEP = 128


def _tc_body(var_ref, ab_ref, out_ref):
    j = pl.program_id(1)

    @pl.when(j == 0)
    def _init():
        out_ref[...] = jnp.zeros_like(out_ref)

    v = var_ref[0]
    a = ab_ref[0]
    d = jnp.where(a != 0, v - a, 0.0)
    dd = d * d
    p = jnp.sum(dd, axis=0).reshape(16, 128).sum(axis=0)
    out_ref[0, 0, :] += p


def _tc_loss(var, ab):
    partial = pl.pallas_call(
        _tc_body,
        grid=(_B, _RT // _TC_BLK),
        in_specs=[
            pl.BlockSpec((1, _TC_BLK, _COLS), lambda b, j: (b, j, 0)),
            pl.BlockSpec((1, _TC_BLK, _COLS), lambda b, j: (b, j, 0)),
        ],
        out_specs=pl.BlockSpec((1, 1, 128), lambda b, j: (b, 0, 0)),
        out_shape=jax.ShapeDtypeStruct((_B, 1, 128), jnp.float32),
    )(var, ab)
    return jnp.sum(partial, axis=(1, 2))


def _chunk_sum(buf_v, buf_a, acc):
    def body(j, acc):
        col = j * _STEP
        for r in range(_CH_ROWS):
            for k in range(_STEP // 16):
                v = buf_v[r, pl.ds(col + k * 16, 16)]
                a = buf_a[r, pl.ds(col + k * 16, 16)]
                d = jnp.where(a == 0.0, a, v) - a
                acc = acc + d * d
        return acc

    return lax.fori_loop(0, _COLS // _STEP, body, acc)


def _sc_loss_body(var_hbm, ab_hbm, out_hbm, vbuf, abuf, obuf, sv0, sv1, sa0, sa1):
    wid = lax.axis_index("s") * 2 + lax.axis_index("c")
    b = wid // _WORKERS_PER_SAMPLE
    row0 = _RT + (wid % _WORKERS_PER_SAMPLE) * _W_ROWS

    sems = (sv0, sv1, sa0, sa1)

    def start(chunk, slot):
        r = row0 + chunk * _CH_ROWS
        pltpu.make_async_copy(var_hbm.at[b, pl.ds(r, _CH_ROWS)], vbuf.at[slot],
                              sems[slot]).start()
        pltpu.make_async_copy(ab_hbm.at[b, pl.ds(r, _CH_ROWS)], abuf.at[slot],
                              sems[2 + slot]).start()

    def wait(chunk, slot):
        r = row0 + chunk * _CH_ROWS
        pltpu.make_async_copy(var_hbm.at[b, pl.ds(r, _CH_ROWS)], vbuf.at[slot],
                              sems[slot]).wait()
        pltpu.make_async_copy(ab_hbm.at[b, pl.ds(r, _CH_ROWS)], abuf.at[slot],
                              sems[2 + slot]).wait()

    start(0, 0)

    def outer(t, acc):
        g0 = 2 * t
        start(g0 + 1, 1)
        wait(g0, 0)
        acc = _chunk_sum(vbuf.at[0], abuf.at[0], acc)

        @pl.when(t + 1 < _NCHUNK // 2)
        def _():
            start(g0 + 2, 0)

        wait(g0 + 1, 1)
        acc = _chunk_sum(vbuf.at[1], abuf.at[1], acc)
        return acc

    acc = lax.fori_loop(0, _NCHUNK // 2, outer, jnp.zeros((16,), jnp.float32))

    zero = jnp.zeros((16,), jnp.float32)
    obuf[pl.ds(0, 16)] = acc
    for k in range(1, 8):
        obuf[pl.ds(k * 16, 16)] = zero
    pltpu.make_async_copy(obuf, out_hbm.at[wid], sv0).start()
    pltpu.make_async_copy(obuf, out_hbm.at[wid], sv0).wait()


_sc_loss = functools.partial(
    pl.kernel,
    mesh=plsc.VectorSubcoreMesh(core_axis_name="c", subcore_axis_name="s"),
    out_type=jax.ShapeDtypeStruct((_NW, 128), jnp.float32),
    scratch_types=[
        pltpu.VMEM((2, _CH_ROWS, _COLS), jnp.float32),
        pltpu.VMEM((2, _CH_ROWS, _COLS), jnp.float32),
        pltpu.VMEM((128,), jnp.float32),
        pltpu.SemaphoreType.DMA,
        pltpu.SemaphoreType.DMA,
        pltpu.SemaphoreType.DMA,
        pltpu.SemaphoreType.DMA,
    ],
    compiler_params=pltpu.CompilerParams(use_tc_tiling_on_sc=True),
)(_sc_loss_body)


def kernel(var, ab):
    sc_partial = _sc_loss(var, ab)
    tc_partial = _tc_loss(var, ab)
    sc = jnp.sum(sc_partial.reshape(_B, _WORKERS_PER_SAMPLE, 128), axis=(1, 2))
    return tc_partial + sc


# speedup vs baseline: 1.0911x; 1.0599x over previous
import jax
import jax.numpy as jnp
from jax.experimental import pallas as pl
from jax.experimental.pallas import tpu as pltpu


_ROWS_PER_BLK = 512


def _loss_body(var_ref, ab_ref, out_ref):
    j = pl.program_id(1)

    @pl.when(j == 0)
    def _init():
        out_ref[...] = jnp.zeros_like(out_ref)

    v = var_ref[0]
    a = ab_ref[0]
    d = jnp.where(a != 0, v - a, 0.0)
    dd = d * d
    p = jnp.sum(dd, axis=0).reshape(16, 128).sum(axis=0)
    out_ref[0, 0, :] += p


def kernel(var, ab):
    B, R, C = var.shape
    nblk = R // _ROWS_PER_BLK
    partial = pl.pallas_call(
        _loss_body,
        grid=(B, nblk),
        in_specs=[
            pl.BlockSpec((1, _ROWS_PER_BLK, C), lambda b, j: (b, j, 0)),
            pl.BlockSpec((1, _ROWS_PER_BLK, C), lambda b, j: (b, j, 0)),
        ],
        out_specs=pl.BlockSpec((1, 1, 128), lambda b, j: (b, 0, 0)),
        out_shape=jax.ShapeDtypeStruct((B, 1, 128), jnp.float32),
    )(var, ab)
    return jnp.sum(partial, axis=(1, 2))
